# Initial kernel scaffold; baseline (speedup 1.0000x reference)
import jax
import jax.numpy as jnp
from jax.experimental import pallas as pl


def kernel(indices, mu_w, sigma_w, eta_w):
    raise NotImplementedError("write your pallas kernel here")


# speedup vs baseline: 1.0512x; 1.0512x over previous
import functools

import jax
import jax.numpy as jnp
from jax import lax
from jax.experimental import pallas as pl
from jax.experimental.pallas import tpu as pltpu
from jax.experimental.pallas import tpu_sc as plsc

N_NODES = 1000000
BATCH = 16384
NUM_CORES = 2
NUM_SUBCORES = 16
NW = NUM_CORES * NUM_SUBCORES
B_PER_W = BATCH // NW
CHUNK = 128
NCHUNK = B_PER_W // CHUNK

_mesh = plsc.VectorSubcoreMesh(core_axis_name="c", subcore_axis_name="s")


@functools.partial(
    pl.kernel,
    mesh=_mesh,
    out_type=[jax.ShapeDtypeStruct((NW, NCHUNK, CHUNK), jnp.float32)] * 3,
    scratch_types=[
        pltpu.VMEM((NCHUNK, CHUNK), jnp.int32),
        pltpu.VMEM((3, NCHUNK, CHUNK), jnp.float32),
        pltpu.SemaphoreType.DMA,
    ],
)
def _gather3(idx_hbm, mu_hbm, sg_hbm, et_hbm, out_mu, out_sg, out_et,
             idx_v, buf_v, sem):
    wid = lax.axis_index("s") * NUM_CORES + lax.axis_index("c")
    pltpu.sync_copy(idx_hbm.at[pl.ds(wid * NCHUNK, NCHUNK)], idx_v)
    copies = []
    for t, tbl in enumerate((mu_hbm, sg_hbm, et_hbm)):
        for j in range(NCHUNK):
            copies.append(
                pltpu.async_copy(tbl.at[idx_v.at[j]], buf_v.at[t, j], sem))
    for c in copies:
        c.wait()
    pltpu.sync_copy(buf_v.at[0], out_mu.at[wid])
    pltpu.sync_copy(buf_v.at[1], out_sg.at[wid])
    pltpu.sync_copy(buf_v.at[2], out_et.at[wid])


def kernel(indices, mu_w, sigma_w, eta_w):
    idx = indices.astype(jnp.int32).reshape(NW * NCHUNK, CHUNK)
    mu, sg, et = (_gather3(idx,
                           mu_w.reshape(-1),
                           sigma_w.reshape(-1),
                           eta_w.reshape(-1)))
    return jnp.stack(
        [mu.reshape(-1), sg.reshape(-1), et.reshape(-1)], axis=-1)
